# baseline (speedup 1.0000x reference)
import functools

import jax
import jax.numpy as jnp
from jax import lax
from jax.experimental import pallas as pl
from jax.experimental.pallas import tpu as pltpu
from jax.experimental.pallas import tpu_sc as plsc

BATCH = 16384
DIM = 32
ROWS = 1000000
NC = 2
NS = 16
NW = NC * NS
BPW = BATCH // NW
WPW = BPW * DIM
CHUNK = 128
NCHUNK = WPW // CHUNK
WINDOW = 8


def _mf_body(u_hbm, i_hbm, uwf_hbm, iwf_hbm, out_hbm,
             uidx_v, iidx_v, uw_idx, iw_idx, ue_v, ie_v, out_v,
             sem_u, sem_i):
    wid = lax.axis_index("s") * NC + lax.axis_index("c")
    base_b = wid * BPW

    pltpu.sync_copy(u_hbm.at[pl.ds(base_b, BPW)], uidx_v)
    pltpu.sync_copy(i_hbm.at[pl.ds(base_b, BPW)], iidx_v)

    def build(g, _):
        b0 = g * 16
        rvec_u = uidx_v[pl.ds(b0, 16)]
        rvec_i = iidx_v[pl.ds(b0, 16)]
        for d in range(DIM):
            uw_idx[pl.ds(d * BPW + b0, 16)] = rvec_u + d * ROWS
            iw_idx[pl.ds(d * BPW + b0, 16)] = rvec_i + d * ROWS
        return _

    lax.fori_loop(0, BPW // 16, build, 0)

    pending = []
    for c in range(NCHUNK):
        o = c * CHUNK
        pending.append(pltpu.async_copy(
            uwf_hbm.at[uw_idx.at[pl.ds(o, CHUNK)]],
            ue_v.at[pl.ds(o, CHUNK)], sem_u))
        pending.append(pltpu.async_copy(
            iwf_hbm.at[iw_idx.at[pl.ds(o, CHUNK)]],
            ie_v.at[pl.ds(o, CHUNK)], sem_i))
        while len(pending) > 2 * WINDOW:
            pending.pop(0).wait()
    while pending:
        pending.pop(0).wait()

    def group(h, _):
        c0 = h * 16
        acc = ue_v[pl.ds(c0, 16)] * ie_v[pl.ds(c0, 16)]
        for d in range(1, DIM):
            o = d * BPW + c0
            acc = acc + ue_v[pl.ds(o, 16)] * ie_v[pl.ds(o, 16)]
        out_v[pl.ds(c0, 16)] = acc
        return _

    lax.fori_loop(0, BPW // 16, group, 0)

    pltpu.sync_copy(out_v, out_hbm.at[pl.ds(base_b, BPW)])


@jax.jit
def _mf_score(u, i, uwf, iwf):
    mesh = plsc.VectorSubcoreMesh(core_axis_name="c", subcore_axis_name="s")
    return pl.kernel(
        _mf_body,
        out_type=jax.ShapeDtypeStruct((BATCH,), jnp.float32),
        mesh=mesh,
        compiler_params=pltpu.CompilerParams(
            needs_layout_passes=False, use_tc_tiling_on_sc=False),
        scratch_types=[
            pltpu.VMEM((BPW,), jnp.int32),
            pltpu.VMEM((BPW,), jnp.int32),
            pltpu.VMEM((WPW,), jnp.int32),
            pltpu.VMEM((WPW,), jnp.int32),
            pltpu.VMEM((WPW,), jnp.float32),
            pltpu.VMEM((WPW,), jnp.float32),
            pltpu.VMEM((BPW,), jnp.float32),
            pltpu.SemaphoreType.DMA,
            pltpu.SemaphoreType.DMA,
        ],
    )(u, i, uwf, iwf)


def kernel(u, i, user_weight, item_weight):
    uwf = user_weight.T.reshape(-1)
    iwf = item_weight.T.reshape(-1)
    return _mf_score(u, i, uwf, iwf)


# speedup vs baseline: 5.6538x; 5.6538x over previous
import functools

import jax
import jax.numpy as jnp
from jax import lax
from jax.experimental import pallas as pl
from jax.experimental.pallas import tpu as pltpu
from jax.experimental.pallas import tpu_sc as plsc

BATCH = 16384
DIM = 32
ROWS = 1000000
GROW = 128
NC = 2
NS = 16
NW = NC * NS
BPW = BATCH // NW
BLK = 256
NBLK = BPW // BLK
CHUNK = 128


def _mf_body(uq_hbm, iq_hbm, uo_hbm, io_hbm, uw_hbm, iw_hbm, out_hbm,
             uq_v, iq_v, uo_v, io_v, ue_v, ie_v, part_v, out_v,
             sem_u, sem_i):
    wid = lax.axis_index("s") * NC + lax.axis_index("c")
    base_b = wid * BPW
    lanes = lax.iota(jnp.int32, 16)

    for blk in range(NBLK):
        b0 = base_b + blk * BLK
        pltpu.sync_copy(uq_hbm.at[pl.ds(b0, BLK)], uq_v)
        pltpu.sync_copy(iq_hbm.at[pl.ds(b0, BLK)], iq_v)
        pltpu.sync_copy(uo_hbm.at[pl.ds(b0, BLK)], uo_v)
        pltpu.sync_copy(io_hbm.at[pl.ds(b0, BLK)], io_v)

        pending = []
        for c in range(BLK // CHUNK):
            o = c * CHUNK
            pending.append(pltpu.async_copy(
                uw_hbm.at[uq_v.at[pl.ds(o, CHUNK)]],
                ue_v.at[pl.ds(o, CHUNK)], sem_u))
            pending.append(pltpu.async_copy(
                iw_hbm.at[iq_v.at[pl.ds(o, CHUNK)]],
                ie_v.at[pl.ds(o, CHUNK)], sem_i))
        for p in pending:
            p.wait()

        def group(g, _):
            gb = g * 16
            um = uo_v[pl.ds(gb, 16)]
            im = io_v[pl.ds(gb, 16)]
            for r in range(16):
                uoff = um[r] * DIM
                ioff = im[r] * DIM
                p = (ue_v[gb + r, pl.ds(uoff, 16)]
                     * ie_v[gb + r, pl.ds(ioff, 16)]
                     + ue_v[gb + r, pl.ds(uoff + 16, 16)]
                     * ie_v[gb + r, pl.ds(ioff + 16, 16)])
                part_v[pl.ds(r * 17, 16)] = p
            acc = plsc.load_gather(part_v, [lanes * 17])
            for l in range(1, 16):
                acc = acc + plsc.load_gather(part_v, [lanes * 17 + l])
            out_v[pl.ds(gb, 16)] = acc
            return _

        lax.fori_loop(0, BLK // 16, group, 0)

        pltpu.sync_copy(out_v, out_hbm.at[pl.ds(b0, BLK)])


@jax.jit
def _mf_score(uq, iq, uo, io, uw4, iw4):
    mesh = plsc.VectorSubcoreMesh(core_axis_name="c", subcore_axis_name="s")
    return pl.kernel(
        _mf_body,
        out_type=jax.ShapeDtypeStruct((BATCH,), jnp.float32),
        mesh=mesh,
        compiler_params=pltpu.CompilerParams(
            needs_layout_passes=False, use_tc_tiling_on_sc=False),
        scratch_types=[
            pltpu.VMEM((BLK,), jnp.int32),
            pltpu.VMEM((BLK,), jnp.int32),
            pltpu.VMEM((BLK,), jnp.int32),
            pltpu.VMEM((BLK,), jnp.int32),
            pltpu.VMEM((BLK, GROW), jnp.float32),
            pltpu.VMEM((BLK, GROW), jnp.float32),
            pltpu.VMEM((16 * 17,), jnp.float32),
            pltpu.VMEM((BLK,), jnp.float32),
            pltpu.SemaphoreType.DMA,
            pltpu.SemaphoreType.DMA,
        ],
    )(uq, iq, uo, io, uw4, iw4)


def kernel(u, i, user_weight, item_weight):
    uw4 = user_weight.reshape(ROWS // 4, GROW)
    iw4 = item_weight.reshape(ROWS // 4, GROW)
    return _mf_score(u >> 2, i >> 2, u & 3, i & 3, uw4, iw4)


# speedup vs baseline: 5.6580x; 1.0007x over previous
import functools

import jax
import jax.numpy as jnp
from jax import lax
from jax.experimental import pallas as pl
from jax.experimental.pallas import tpu as pltpu
from jax.experimental.pallas import tpu_sc as plsc

BATCH = 16384
DIM = 32
ROWS = 1000000
GROW = 128
NC = 2
NS = 16
NW = NC * NS
BPW = BATCH // NW
BLK = 256
NBLK = BPW // BLK
CHUNK = 128


def _mf_body(uq_hbm, iq_hbm, uo_hbm, io_hbm, uw_hbm, iw_hbm, out_hbm,
             uq_v, iq_v, uo_v, io_v, ue_v, ie_v, part_v, out_v,
             sem_u, sem_i):
    wid = lax.axis_index("s") * NC + lax.axis_index("c")
    base_b = wid * BPW
    lanes = lax.iota(jnp.int32, 16)

    for blk in range(NBLK):
        b0 = base_b + blk * BLK
        pltpu.sync_copy(uq_hbm.at[pl.ds(b0, BLK)], uq_v)
        pltpu.sync_copy(iq_hbm.at[pl.ds(b0, BLK)], iq_v)
        pltpu.sync_copy(uo_hbm.at[pl.ds(b0, BLK)], uo_v)
        pltpu.sync_copy(io_hbm.at[pl.ds(b0, BLK)], io_v)

        pending = []
        for c in range(BLK // CHUNK):
            o = c * CHUNK
            pending.append(pltpu.async_copy(
                uw_hbm.at[uq_v.at[pl.ds(o, CHUNK)]],
                ue_v.at[pl.ds(o, CHUNK)], sem_u))
            pending.append(pltpu.async_copy(
                iw_hbm.at[iq_v.at[pl.ds(o, CHUNK)]],
                ie_v.at[pl.ds(o, CHUNK)], sem_i))
        for p in pending:
            p.wait()

        def group(g, _):
            gb = g * 16
            um = uo_v[pl.ds(gb, 16)]
            im = io_v[pl.ds(gb, 16)]
            for r in range(16):
                uoff = um[r] * DIM
                ioff = im[r] * DIM
                p = (ue_v[gb + r, pl.ds(uoff, 16)]
                     * ie_v[gb + r, pl.ds(ioff, 16)]
                     + ue_v[gb + r, pl.ds(uoff + 16, 16)]
                     * ie_v[gb + r, pl.ds(ioff + 16, 16)])
                part_v[pl.ds(r * 17, 16)] = p
            acc = plsc.load_gather(part_v, [lanes * 17])
            for l in range(1, 16):
                acc = acc + plsc.load_gather(part_v, [lanes * 17 + l])
            out_v[pl.ds(gb, 16)] = acc
            return _

        lax.fori_loop(0, BLK // 16, group, 0)

        pltpu.sync_copy(out_v, out_hbm.at[pl.ds(b0, BLK)])


@jax.jit
def _mf_score(uq, iq, uo, io, uw4, iw4):
    mesh = plsc.VectorSubcoreMesh(core_axis_name="c", subcore_axis_name="s")
    return pl.kernel(
        _mf_body,
        out_type=jax.ShapeDtypeStruct((BATCH,), jnp.float32),
        mesh=mesh,
        compiler_params=pltpu.CompilerParams(
            needs_layout_passes=False, use_tc_tiling_on_sc=True),
        scratch_types=[
            pltpu.VMEM((BLK,), jnp.int32),
            pltpu.VMEM((BLK,), jnp.int32),
            pltpu.VMEM((BLK,), jnp.int32),
            pltpu.VMEM((BLK,), jnp.int32),
            pltpu.VMEM((BLK, GROW), jnp.float32),
            pltpu.VMEM((BLK, GROW), jnp.float32),
            pltpu.VMEM((16 * 17,), jnp.float32),
            pltpu.VMEM((BLK,), jnp.float32),
            pltpu.SemaphoreType.DMA,
            pltpu.SemaphoreType.DMA,
        ],
    )(uq, iq, uo, io, uw4, iw4)


def kernel(u, i, user_weight, item_weight):
    uw4 = user_weight.reshape(ROWS // 4, GROW)
    iw4 = item_weight.reshape(ROWS // 4, GROW)
    return _mf_score(u >> 2, i >> 2, u & 3, i & 3, uw4, iw4)


# speedup vs baseline: 5.7039x; 1.0081x over previous
import functools

import jax
import jax.numpy as jnp
from jax import lax
from jax.experimental import pallas as pl
from jax.experimental.pallas import tpu as pltpu
from jax.experimental.pallas import tpu_sc as plsc

BATCH = 16384
DIM = 32
NC = 2
NS = 16
NW = NC * NS
BPW = BATCH // NW
CHUNK = 128
NCHUNK = BPW // CHUNK


def _mf_body(u_hbm, i_hbm, uw_hbm, iw_hbm, out_hbm,
             uidx_v, iidx_v, ue_v, ie_v, part_v, out_v, sem_u, sem_i):
    wid = lax.axis_index("s") * NC + lax.axis_index("c")

    pltpu.sync_copy(u_hbm.at[wid], uidx_v)
    pltpu.sync_copy(i_hbm.at[wid], iidx_v)

    copies = []
    for j in range(NCHUNK):
        copies.append(pltpu.async_copy(
            uw_hbm.at[uidx_v.at[j]], ue_v.at[pl.ds(j * CHUNK, CHUNK)], sem_u))
        copies.append(pltpu.async_copy(
            iw_hbm.at[iidx_v.at[j]], ie_v.at[pl.ds(j * CHUNK, CHUNK)], sem_i))
    for c in copies:
        c.wait()

    lanes = lax.iota(jnp.int32, 16)

    def group(g, _):
        base = g * 16
        for r in range(16):
            b = base + r
            p = (ue_v[b, pl.ds(0, 16)] * ie_v[b, pl.ds(0, 16)]
                 + ue_v[b, pl.ds(16, 16)] * ie_v[b, pl.ds(16, 16)])
            part_v[r, pl.ds(0, 16)] = p
        acc = plsc.load_gather(part_v, [lanes, jnp.zeros((16,), jnp.int32)])
        for l in range(1, 16):
            acc = acc + plsc.load_gather(
                part_v, [lanes, jnp.full((16,), l, jnp.int32)])
        out_v[pl.ds(base, 16)] = acc
        return _

    lax.fori_loop(0, BPW // 16, group, 0)

    pltpu.sync_copy(out_v, out_hbm.at[wid])


@jax.jit
def _mf_score(u2, i2, user_weight, item_weight):
    mesh = plsc.VectorSubcoreMesh(core_axis_name="c", subcore_axis_name="s")
    return pl.kernel(
        _mf_body,
        out_type=jax.ShapeDtypeStruct((NW, BPW), jnp.float32),
        mesh=mesh,
        compiler_params=pltpu.CompilerParams(
            needs_layout_passes=False, use_tc_tiling_on_sc=False),
        scratch_types=[
            pltpu.VMEM((NCHUNK, CHUNK), jnp.int32),
            pltpu.VMEM((NCHUNK, CHUNK), jnp.int32),
            pltpu.VMEM((BPW, DIM), jnp.float32),
            pltpu.VMEM((BPW, DIM), jnp.float32),
            pltpu.VMEM((16, 17), jnp.float32),
            pltpu.VMEM((BPW,), jnp.float32),
            pltpu.SemaphoreType.DMA,
            pltpu.SemaphoreType.DMA,
        ],
    )(u2, i2, user_weight, item_weight)


def kernel(u, i, user_weight, item_weight):
    u2 = u.reshape(NW, NCHUNK, CHUNK)
    i2 = i.reshape(NW, NCHUNK, CHUNK)
    out = _mf_score(u2, i2, user_weight, item_weight)
    return out.reshape(BATCH)
